# baseline (speedup 1.0000x reference)
import functools

import jax
import jax.numpy as jnp
from jax import lax
from jax.experimental import pallas as pl
from jax.experimental.pallas import tpu as pltpu
from jax.experimental.pallas import tpu_sc as plsc

_B = 16
_C = 256
_H = 256
_KOUT = 18
_N = 262144

_NC = 2
_NS = 16
_NW = _NC * _NS
_RPW = _N // _NW
_CHUNK = 128
_NCHUNK = _RPW // _CHUNK


def _sc_segsum_body(feat_hbm, offset_hbm, out_hbm, buf, acc, offs_smem, sem):
    wid = lax.axis_index("s") * _NC + lax.axis_index("c")
    base = wid * _RPW

    pltpu.sync_copy(offset_hbm, offs_smem)
    offs = offs_smem[...]
    ends = [offs[j] for j in range(_B)]
    starts = [jnp.int32(0)] + ends[:-1]

    zero16 = jnp.zeros((16,), jnp.float32)
    for j in range(_B):
        for c in range(_C // 16):
            acc[j, pl.ds(c * 16, 16)] = zero16

    nlane = _C // 16

    def chunk_body(k, carry):
        pltpu.async_copy(
            feat_hbm.at[pl.ds(base + k * _CHUNK, _CHUNK)], buf, sem).wait()
        cb = base + k * _CHUNK

        for j in range(_B):
            lo = jnp.clip(starts[j] - cb, 0, _CHUNK)
            hi = jnp.clip(ends[j] - cb, 0, _CHUNK)

            @pl.when(hi > lo)
            def _seg(j=j, lo=lo, hi=hi):
                def row_body(r, c2):
                    for c in range(nlane):
                        plsc.addupdate(acc.at[j, pl.ds(c * 16, 16)],
                                       buf[r, pl.ds(c * 16, 16)])
                    return c2

                lax.fori_loop(lo, hi, row_body, 0)

        return carry

    lax.fori_loop(0, _NCHUNK, chunk_body, 0)

    pltpu.sync_copy(acc, out_hbm.at[wid])


def _sc_segsum(feat, offset):
    mesh = plsc.VectorSubcoreMesh(core_axis_name="c", subcore_axis_name="s")
    return pl.kernel(
        _sc_segsum_body,
        mesh=mesh,
        out_type=jax.ShapeDtypeStruct((_NW, _B, _C), jnp.float32),
        scratch_types=[
            pltpu.VMEM((_CHUNK, _C), jnp.float32),
            pltpu.VMEM((_B, _C), jnp.float32),
            pltpu.VMEM((_B,), jnp.int32),
            pltpu.SemaphoreType.DMA,
        ],
    )(feat, offset)


def _head_body(part_ref, invc_ref, w1_ref, bn1s_ref, bn1b_ref,
               w2_ref, b2_ref, w3_ref, b3_ref, out_ref):
    rows = lax.broadcasted_iota(jnp.int32, (_B, _NW * _B), 1)
    cols = lax.broadcasted_iota(jnp.int32, (_B, _NW * _B), 0)
    sel = (rows % _B == cols).astype(jnp.float32)
    seg_sum = jnp.dot(sel, part_ref[...], preferred_element_type=jnp.float32)
    gf = seg_sum * invc_ref[...]
    h = jnp.dot(gf, w1_ref[...], preferred_element_type=jnp.float32)
    h = h * bn1s_ref[...] + bn1b_ref[...]
    h = jnp.maximum(h, 0.0)
    h = jnp.dot(h, w2_ref[...], preferred_element_type=jnp.float32) + b2_ref[...]
    h = jnp.maximum(h, 0.0)
    out_ref[...] = (jnp.dot(h, w3_ref[...], preferred_element_type=jnp.float32)
                    + b3_ref[...])


def kernel(feat, offset, W1, b1, gamma, beta, rmean, rvar, W2, b2, W3, b3):
    offset = offset.astype(jnp.int32)
    starts = jnp.concatenate([jnp.zeros((1,), jnp.int32), offset[:-1]])
    counts = jnp.maximum((offset - starts).astype(jnp.float32), 1.0)
    invc = (1.0 / counts).reshape(_B, 1)
    bn1s = gamma * lax.rsqrt(rvar + 1e-5)
    bn1b = (b1 - rmean) * bn1s + beta

    partials = _sc_segsum(feat, offset).reshape(_NW * _B, _C)

    out = pl.pallas_call(
        _head_body,
        in_specs=[
            pl.BlockSpec((_NW * _B, _C), lambda: (0, 0)),
            pl.BlockSpec((_B, 1), lambda: (0, 0)),
            pl.BlockSpec((_C, _H), lambda: (0, 0)),
            pl.BlockSpec((1, _H), lambda: (0, 0)),
            pl.BlockSpec((1, _H), lambda: (0, 0)),
            pl.BlockSpec((_H, _H), lambda: (0, 0)),
            pl.BlockSpec((1, _H), lambda: (0, 0)),
            pl.BlockSpec((_H, _KOUT), lambda: (0, 0)),
            pl.BlockSpec((1, _KOUT), lambda: (0, 0)),
        ],
        out_specs=pl.BlockSpec((_B, _KOUT), lambda: (0, 0)),
        out_shape=jax.ShapeDtypeStruct((_B, _KOUT), jnp.float32),
    )(partials, invc,
      W1, bn1s.reshape(1, _H), bn1b.reshape(1, _H),
      W2, b2.reshape(1, _H), W3, b3.reshape(1, _KOUT))
    return out.reshape(_B, 6, 3)


# speedup vs baseline: 1.7696x; 1.7696x over previous
import functools

import jax
import jax.numpy as jnp
from jax import lax
from jax.experimental import pallas as pl
from jax.experimental.pallas import tpu as pltpu
from jax.experimental.pallas import tpu_sc as plsc

_B = 16
_C = 256
_H = 256
_KOUT = 18
_N = 262144

_NC = 2
_NS = 16
_NW = _NC * _NS
_RPW = _N // _NW
_CHUNK = 128
_NCHUNK = _RPW // _CHUNK
_G = 8


def _sc_segsum_body(feat_hbm, offset_hbm, out_hbm, buf, acc, cacc, offs_smem, sem):
    wid = lax.axis_index("s") * _NC + lax.axis_index("c")
    base = wid * _RPW

    pltpu.sync_copy(offset_hbm, offs_smem)
    offs = offs_smem[...]
    ends = [offs[j] for j in range(_B)]
    starts = [jnp.int32(0)] + ends[:-1]

    zero16 = jnp.zeros((16,), jnp.float32)
    for j in range(_B):
        for c in range(_C // 16):
            acc[j, pl.ds(c * 16, 16)] = zero16

    nlane = _C // 16

    def chunk_body(k, carry):
        pltpu.async_copy(
            feat_hbm.at[pl.ds(base + k * _CHUNK, _CHUNK)], buf, sem).wait()
        cb = base + k * _CHUNK

        sf = sum(((ends[j] <= cb).astype(jnp.int32)) for j in range(_B))
        sl = sum(((ends[j] <= cb + _CHUNK - 1).astype(jnp.int32))
                 for j in range(_B))

        @pl.when(sf == sl)
        def _fast():
            for c in range(nlane):
                cacc[pl.ds(c * 16, 16)] = zero16

            def grp_body(g, c2):
                rb = g * _G
                for c in range(nlane):
                    s = buf[rb, pl.ds(c * 16, 16)]
                    for t in range(1, _G):
                        s = s + buf[rb + t, pl.ds(c * 16, 16)]
                    plsc.addupdate(cacc.at[pl.ds(c * 16, 16)], s)
                return c2

            lax.fori_loop(0, _CHUNK // _G, grp_body, 0)
            for j in range(_B):
                @pl.when(sf == j)
                def _flush(j=j):
                    for c in range(nlane):
                        plsc.addupdate(acc.at[j, pl.ds(c * 16, 16)],
                                       cacc[pl.ds(c * 16, 16)])

        @pl.when(sf != sl)
        def _slow():
            for j in range(_B):
                lo = jnp.clip(starts[j] - cb, 0, _CHUNK)
                hi = jnp.clip(ends[j] - cb, 0, _CHUNK)

                @pl.when(hi > lo)
                def _seg(j=j, lo=lo, hi=hi):
                    def rem_body(r, c2):
                        for c in range(nlane):
                            plsc.addupdate(acc.at[j, pl.ds(c * 16, 16)],
                                           buf[r, pl.ds(c * 16, 16)])
                        return c2

                    lax.fori_loop(lo, hi, rem_body, 0)

        return carry

    lax.fori_loop(0, _NCHUNK, chunk_body, 0)

    pltpu.sync_copy(acc, out_hbm.at[wid])


def _sc_segsum(feat, offset):
    mesh = plsc.VectorSubcoreMesh(core_axis_name="c", subcore_axis_name="s")
    return pl.kernel(
        _sc_segsum_body,
        mesh=mesh,
        out_type=jax.ShapeDtypeStruct((_NW, _B, _C), jnp.float32),
        scratch_types=[
            pltpu.VMEM((_CHUNK, _C), jnp.float32),
            pltpu.VMEM((_B, _C), jnp.float32),
            pltpu.VMEM((_C,), jnp.float32),
            pltpu.VMEM((_B,), jnp.int32),
            pltpu.SemaphoreType.DMA,
        ],
    )(feat, offset)


def _head_body(part_ref, invc_ref, w1_ref, bn1s_ref, bn1b_ref,
               w2_ref, b2_ref, w3_ref, b3_ref, out_ref):
    rows = lax.broadcasted_iota(jnp.int32, (_B, _NW * _B), 1)
    cols = lax.broadcasted_iota(jnp.int32, (_B, _NW * _B), 0)
    sel = (rows % _B == cols).astype(jnp.float32)
    seg_sum = jnp.dot(sel, part_ref[...], preferred_element_type=jnp.float32)
    gf = seg_sum * invc_ref[...]
    h = jnp.dot(gf, w1_ref[...], preferred_element_type=jnp.float32)
    h = h * bn1s_ref[...] + bn1b_ref[...]
    h = jnp.maximum(h, 0.0)
    h = jnp.dot(h, w2_ref[...], preferred_element_type=jnp.float32) + b2_ref[...]
    h = jnp.maximum(h, 0.0)
    out_ref[...] = (jnp.dot(h, w3_ref[...], preferred_element_type=jnp.float32)
                    + b3_ref[...])


def kernel(feat, offset, W1, b1, gamma, beta, rmean, rvar, W2, b2, W3, b3):
    offset = offset.astype(jnp.int32)
    starts = jnp.concatenate([jnp.zeros((1,), jnp.int32), offset[:-1]])
    counts = jnp.maximum((offset - starts).astype(jnp.float32), 1.0)
    invc = (1.0 / counts).reshape(_B, 1)
    bn1s = gamma * lax.rsqrt(rvar + 1e-5)
    bn1b = (b1 - rmean) * bn1s + beta

    partials = _sc_segsum(feat, offset).reshape(_NW * _B, _C)

    out = pl.pallas_call(
        _head_body,
        in_specs=[
            pl.BlockSpec((_NW * _B, _C), lambda: (0, 0)),
            pl.BlockSpec((_B, 1), lambda: (0, 0)),
            pl.BlockSpec((_C, _H), lambda: (0, 0)),
            pl.BlockSpec((1, _H), lambda: (0, 0)),
            pl.BlockSpec((1, _H), lambda: (0, 0)),
            pl.BlockSpec((_H, _H), lambda: (0, 0)),
            pl.BlockSpec((1, _H), lambda: (0, 0)),
            pl.BlockSpec((_H, _KOUT), lambda: (0, 0)),
            pl.BlockSpec((1, _KOUT), lambda: (0, 0)),
        ],
        out_specs=pl.BlockSpec((_B, _KOUT), lambda: (0, 0)),
        out_shape=jax.ShapeDtypeStruct((_B, _KOUT), jnp.float32),
    )(partials, invc,
      W1, bn1s.reshape(1, _H), bn1b.reshape(1, _H),
      W2, b2.reshape(1, _H), W3, b3.reshape(1, _KOUT))
    return out.reshape(_B, 6, 3)


# speedup vs baseline: 3.8674x; 2.1854x over previous
import jax
import jax.numpy as jnp
from jax import lax
from jax.experimental import pallas as pl
from jax.experimental.pallas import tpu as pltpu
from jax.experimental.pallas import tpu_sc as plsc

_B = 16
_C = 256
_H = 256
_KOUT = 18
_N = 262144

_NC = 2
_NS = 16
_NW = _NC * _NS
_RPW = _N // _NW
_CHUNK = 128
_NCHUNK = _RPW // _CHUNK
_G = 8
_NLANE = _C // 16


def _sc_segsum_body(feat_hbm, offset_hbm, out_hbm, buf_a, buf_b, acc,
                    offs_v, sem_a, sem_b):
    wid = lax.axis_index("s") * _NC + lax.axis_index("c")
    base = wid * _RPW

    pltpu.sync_copy(offset_hbm, offs_v)
    offs = offs_v[...]
    ends = [offs[j] for j in range(_B)]
    starts = [jnp.int32(0)] + ends[:-1]

    zero16 = jnp.zeros((16,), jnp.float32)
    for j in range(_B):
        for c in range(_NLANE):
            acc[j, pl.ds(c * 16, 16)] = zero16

    def issue(k, buf, sem):
        pltpu.async_copy(
            feat_hbm.at[pl.ds(base + k * _CHUNK, _CHUNK)], buf, sem)

    def wait(k, buf, sem):
        pltpu.make_async_copy(
            feat_hbm.at[pl.ds(base + k * _CHUNK, _CHUNK)], buf, sem).wait()

    def process(buf, k):
        cb = base + k * _CHUNK
        sf = sum(((ends[j] <= cb).astype(jnp.int32)) for j in range(_B))
        sl = sum(((ends[j] <= cb + _CHUNK - 1).astype(jnp.int32))
                 for j in range(_B))

        @pl.when(sf == sl)
        def _fast():
            init = tuple(zero16 for _ in range(_NLANE))

            def grp_body(g, vecs):
                rb = g * _G
                out = []
                for c in range(_NLANE):
                    s = vecs[c]
                    for t in range(_G):
                        s = s + buf[rb + t, pl.ds(c * 16, 16)]
                    out.append(s)
                return tuple(out)

            vecs = plsc.parallel_loop(
                0, _CHUNK // _G, carry=init)(grp_body)
            for j in range(_B):
                @pl.when(sf == j)
                def _flush(j=j, vecs=vecs):
                    for c in range(_NLANE):
                        plsc.addupdate(acc.at[j, pl.ds(c * 16, 16)], vecs[c])

        @pl.when(sf != sl)
        def _slow():
            for j in range(_B):
                lo = jnp.clip(starts[j] - cb, 0, _CHUNK)
                hi = jnp.clip(ends[j] - cb, 0, _CHUNK)

                @pl.when(hi > lo)
                def _seg(j=j, lo=lo, hi=hi):
                    def rem_body(r, c2):
                        for c in range(_NLANE):
                            plsc.addupdate(acc.at[j, pl.ds(c * 16, 16)],
                                           buf[r, pl.ds(c * 16, 16)])
                        return c2

                    lax.fori_loop(lo, hi, rem_body, 0)

    issue(0, buf_a, sem_a)
    issue(1, buf_b, sem_b)

    def pair_body(p, carry):
        k0 = 2 * p
        wait(k0, buf_a, sem_a)
        process(buf_a, k0)

        @pl.when(k0 + 2 < _NCHUNK)
        def _ra():
            issue(k0 + 2, buf_a, sem_a)

        wait(k0 + 1, buf_b, sem_b)
        process(buf_b, k0 + 1)

        @pl.when(k0 + 3 < _NCHUNK)
        def _rb():
            issue(k0 + 3, buf_b, sem_b)

        return carry

    lax.fori_loop(0, _NCHUNK // 2, pair_body, 0)

    pltpu.sync_copy(acc, out_hbm.at[wid])


def _sc_segsum(feat, offset):
    mesh = plsc.VectorSubcoreMesh(core_axis_name="c", subcore_axis_name="s")
    return pl.kernel(
        _sc_segsum_body,
        mesh=mesh,
        out_type=jax.ShapeDtypeStruct((_NW, _B, _C), jnp.float32),
        scratch_types=[
            pltpu.VMEM((_CHUNK, _C), jnp.float32),
            pltpu.VMEM((_CHUNK, _C), jnp.float32),
            pltpu.VMEM((_B, _C), jnp.float32),
            pltpu.VMEM((_B,), jnp.int32),
            pltpu.SemaphoreType.DMA,
            pltpu.SemaphoreType.DMA,
        ],
    )(feat, offset)


def _head_body(part_ref, invc_ref, w1_ref, bn1s_ref, bn1b_ref,
               w2_ref, b2_ref, w3_ref, b3_ref, out_ref):
    rows = lax.broadcasted_iota(jnp.int32, (_B, _NW * _B), 1)
    cols = lax.broadcasted_iota(jnp.int32, (_B, _NW * _B), 0)
    sel = (rows % _B == cols).astype(jnp.float32)
    seg_sum = jnp.dot(sel, part_ref[...], preferred_element_type=jnp.float32)
    gf = seg_sum * invc_ref[...]
    h = jnp.dot(gf, w1_ref[...], preferred_element_type=jnp.float32)
    h = h * bn1s_ref[...] + bn1b_ref[...]
    h = jnp.maximum(h, 0.0)
    h = jnp.dot(h, w2_ref[...], preferred_element_type=jnp.float32) + b2_ref[...]
    h = jnp.maximum(h, 0.0)
    out_ref[...] = (jnp.dot(h, w3_ref[...], preferred_element_type=jnp.float32)
                    + b3_ref[...])


def kernel(feat, offset, W1, b1, gamma, beta, rmean, rvar, W2, b2, W3, b3):
    offset = offset.astype(jnp.int32)
    starts = jnp.concatenate([jnp.zeros((1,), jnp.int32), offset[:-1]])
    counts = jnp.maximum((offset - starts).astype(jnp.float32), 1.0)
    invc = (1.0 / counts).reshape(_B, 1)
    bn1s = gamma * lax.rsqrt(rvar + 1e-5)
    bn1b = (b1 - rmean) * bn1s + beta

    partials = _sc_segsum(feat, offset).reshape(_NW * _B, _C)

    out = pl.pallas_call(
        _head_body,
        in_specs=[
            pl.BlockSpec((_NW * _B, _C), lambda: (0, 0)),
            pl.BlockSpec((_B, 1), lambda: (0, 0)),
            pl.BlockSpec((_C, _H), lambda: (0, 0)),
            pl.BlockSpec((1, _H), lambda: (0, 0)),
            pl.BlockSpec((1, _H), lambda: (0, 0)),
            pl.BlockSpec((_H, _H), lambda: (0, 0)),
            pl.BlockSpec((1, _H), lambda: (0, 0)),
            pl.BlockSpec((_H, _KOUT), lambda: (0, 0)),
            pl.BlockSpec((1, _KOUT), lambda: (0, 0)),
        ],
        out_specs=pl.BlockSpec((_B, _KOUT), lambda: (0, 0)),
        out_shape=jax.ShapeDtypeStruct((_B, _KOUT), jnp.float32),
    )(partials, invc,
      W1, bn1s.reshape(1, _H), bn1b.reshape(1, _H),
      W2, b2.reshape(1, _H), W3, b3.reshape(1, _KOUT))
    return out.reshape(_B, 6, 3)


# speedup vs baseline: 6.0299x; 1.5591x over previous
import jax
import jax.numpy as jnp
from jax import lax
from jax.experimental import pallas as pl
from jax.experimental.pallas import tpu as pltpu
from jax.experimental.pallas import tpu_sc as plsc

_B = 16
_C = 256
_H = 256
_KOUT = 18
_N = 262144

_NC = 2
_NS = 16
_NW = _NC * _NS
_S_SC = 90112
_RPW = _S_SC // _NW
_CHUNK = 128
_NCHUNK = _RPW // _CHUNK
_G = 8
_NLANE = _C // 16
_TBLK = 8192


def _sc_segsum_body(feat_hbm, offset_hbm, out_hbm, buf_a, buf_b, acc,
                    offs_v, sem_a, sem_b):
    wid = lax.axis_index("s") * _NC + lax.axis_index("c")
    base = wid * _RPW

    pltpu.sync_copy(offset_hbm, offs_v)
    offs = offs_v[...]
    ends = [offs[j] for j in range(_B)]
    starts = [jnp.int32(0)] + ends[:-1]

    zero16 = jnp.zeros((16,), jnp.float32)
    for j in range(_B):
        for c in range(_NLANE):
            acc[j, pl.ds(c * 16, 16)] = zero16

    def issue(k, buf, sem):
        pltpu.async_copy(
            feat_hbm.at[pl.ds(base + k * _CHUNK, _CHUNK)], buf, sem)

    def wait(k, buf, sem):
        pltpu.make_async_copy(
            feat_hbm.at[pl.ds(base + k * _CHUNK, _CHUNK)], buf, sem).wait()

    def process(buf, k):
        cb = base + k * _CHUNK
        sf = sum(((ends[j] <= cb).astype(jnp.int32)) for j in range(_B))
        sl = sum(((ends[j] <= cb + _CHUNK - 1).astype(jnp.int32))
                 for j in range(_B))

        @pl.when(sf == sl)
        def _fast():
            init = tuple(zero16 for _ in range(_NLANE))

            def grp_body(g, vecs):
                rb = g * _G
                out = []
                for c in range(_NLANE):
                    s = vecs[c]
                    for t in range(_G):
                        s = s + buf[rb + t, pl.ds(c * 16, 16)]
                    out.append(s)
                return tuple(out)

            vecs = plsc.parallel_loop(
                0, _CHUNK // _G, carry=init)(grp_body)
            for j in range(_B):
                @pl.when(sf == j)
                def _flush(j=j, vecs=vecs):
                    for c in range(_NLANE):
                        plsc.addupdate(acc.at[j, pl.ds(c * 16, 16)], vecs[c])

        @pl.when(sf != sl)
        def _slow():
            for j in range(_B):
                lo = jnp.clip(starts[j] - cb, 0, _CHUNK)
                hi = jnp.clip(ends[j] - cb, 0, _CHUNK)

                @pl.when(hi > lo)
                def _seg(j=j, lo=lo, hi=hi):
                    def rem_body(r, c2):
                        for c in range(_NLANE):
                            plsc.addupdate(acc.at[j, pl.ds(c * 16, 16)],
                                           buf[r, pl.ds(c * 16, 16)])
                        return c2

                    lax.fori_loop(lo, hi, rem_body, 0)

    issue(0, buf_a, sem_a)
    issue(1, buf_b, sem_b)

    def pair_body(p, carry):
        k0 = 2 * p
        wait(k0, buf_a, sem_a)
        process(buf_a, k0)

        @pl.when(k0 + 2 < _NCHUNK)
        def _ra():
            issue(k0 + 2, buf_a, sem_a)

        wait(k0 + 1, buf_b, sem_b)
        process(buf_b, k0 + 1)

        @pl.when(k0 + 3 < _NCHUNK)
        def _rb():
            issue(k0 + 3, buf_b, sem_b)

        return carry

    lax.fori_loop(0, _NCHUNK // 2, pair_body, 0)

    pltpu.sync_copy(acc, out_hbm.at[wid])


def _sc_segsum(feat, offset):
    mesh = plsc.VectorSubcoreMesh(core_axis_name="c", subcore_axis_name="s")
    return pl.kernel(
        _sc_segsum_body,
        mesh=mesh,
        out_type=jax.ShapeDtypeStruct((_NW, _B, _C), jnp.float32),
        scratch_types=[
            pltpu.VMEM((_CHUNK, _C), jnp.float32),
            pltpu.VMEM((_CHUNK, _C), jnp.float32),
            pltpu.VMEM((_B, _C), jnp.float32),
            pltpu.VMEM((_B,), jnp.int32),
            pltpu.SemaphoreType.DMA,
            pltpu.SemaphoreType.DMA,
        ],
    )(feat, offset)


def _tc_partial_body(feat_ref, starts_ref, ends_ref, out_ref, *, nsteps):
    i = pl.program_id(0)

    @pl.when(i == 0)
    def _init():
        out_ref[...] = jnp.zeros_like(out_ref)

    rows = (lax.broadcasted_iota(jnp.int32, (_B, _TBLK), 1)
            + (i + _S_SC // _TBLK) * _TBLK)
    onehot_t = ((rows >= starts_ref[...]) & (rows < ends_ref[...])
                ).astype(jnp.float32)
    out_ref[...] += jnp.dot(onehot_t, feat_ref[...],
                            preferred_element_type=jnp.float32)


def _tc_partial(feat, starts2d, ends2d):
    nsteps = (_N - _S_SC) // _TBLK
    import functools
    body = functools.partial(_tc_partial_body, nsteps=nsteps)
    return pl.pallas_call(
        body,
        grid=(nsteps,),
        in_specs=[
            pl.BlockSpec((_TBLK, _C), lambda i: (i + _S_SC // _TBLK, 0)),
            pl.BlockSpec((_B, 1), lambda i: (0, 0)),
            pl.BlockSpec((_B, 1), lambda i: (0, 0)),
        ],
        out_specs=pl.BlockSpec((_B, _C), lambda i: (0, 0)),
        out_shape=jax.ShapeDtypeStruct((_B, _C), jnp.float32),
        compiler_params=pltpu.CompilerParams(
            dimension_semantics=("arbitrary",)),
    )(feat, starts2d, ends2d)


def _head_body(part_ref, tcp_ref, invc_ref, w1_ref, bn1s_ref, bn1b_ref,
               w2_ref, b2_ref, w3_ref, b3_ref, out_ref):
    rows = lax.broadcasted_iota(jnp.int32, (_B, _NW * _B), 1)
    cols = lax.broadcasted_iota(jnp.int32, (_B, _NW * _B), 0)
    sel = (rows % _B == cols).astype(jnp.float32)
    seg_sum = (jnp.dot(sel, part_ref[...], preferred_element_type=jnp.float32)
               + tcp_ref[...])
    gf = seg_sum * invc_ref[...]
    h = jnp.dot(gf, w1_ref[...], preferred_element_type=jnp.float32)
    h = h * bn1s_ref[...] + bn1b_ref[...]
    h = jnp.maximum(h, 0.0)
    h = jnp.dot(h, w2_ref[...], preferred_element_type=jnp.float32) + b2_ref[...]
    h = jnp.maximum(h, 0.0)
    out_ref[...] = (jnp.dot(h, w3_ref[...], preferred_element_type=jnp.float32)
                    + b3_ref[...])


def kernel(feat, offset, W1, b1, gamma, beta, rmean, rvar, W2, b2, W3, b3):
    offset = offset.astype(jnp.int32)
    starts = jnp.concatenate([jnp.zeros((1,), jnp.int32), offset[:-1]])
    counts = jnp.maximum((offset - starts).astype(jnp.float32), 1.0)
    invc = (1.0 / counts).reshape(_B, 1)
    bn1s = gamma * lax.rsqrt(rvar + 1e-5)
    bn1b = (b1 - rmean) * bn1s + beta

    partials = _sc_segsum(feat, offset).reshape(_NW * _B, _C)
    tcp = _tc_partial(feat, starts.reshape(_B, 1), offset.reshape(_B, 1))

    out = pl.pallas_call(
        _head_body,
        in_specs=[
            pl.BlockSpec((_NW * _B, _C), lambda: (0, 0)),
            pl.BlockSpec((_B, _C), lambda: (0, 0)),
            pl.BlockSpec((_B, 1), lambda: (0, 0)),
            pl.BlockSpec((_C, _H), lambda: (0, 0)),
            pl.BlockSpec((1, _H), lambda: (0, 0)),
            pl.BlockSpec((1, _H), lambda: (0, 0)),
            pl.BlockSpec((_H, _H), lambda: (0, 0)),
            pl.BlockSpec((1, _H), lambda: (0, 0)),
            pl.BlockSpec((_H, _KOUT), lambda: (0, 0)),
            pl.BlockSpec((1, _KOUT), lambda: (0, 0)),
        ],
        out_specs=pl.BlockSpec((_B, _KOUT), lambda: (0, 0)),
        out_shape=jax.ShapeDtypeStruct((_B, _KOUT), jnp.float32),
    )(partials, tcp, invc,
      W1, bn1s.reshape(1, _H), bn1b.reshape(1, _H),
      W2, b2.reshape(1, _H), W3, b3.reshape(1, _KOUT))
    return out.reshape(_B, 6, 3)


# speedup vs baseline: 6.0970x; 1.0111x over previous
import jax
import jax.numpy as jnp
from jax import lax
from jax.experimental import pallas as pl
from jax.experimental.pallas import tpu as pltpu
from jax.experimental.pallas import tpu_sc as plsc

_B = 16
_C = 256
_H = 256
_KOUT = 18
_N = 262144

_NC = 2
_NS = 16
_NW = _NC * _NS
_S_SC = 65536
_RPW = _S_SC // _NW
_CHUNK = 128
_NCHUNK = _RPW // _CHUNK
_G = 8
_NLANE = _C // 16
_TBLK = 8192


def _sc_segsum_body(feat_hbm, offset_hbm, out_hbm, buf_a, buf_b, acc,
                    offs_v, sem_a, sem_b):
    wid = lax.axis_index("s") * _NC + lax.axis_index("c")
    base = wid * _RPW

    pltpu.sync_copy(offset_hbm, offs_v)
    offs = offs_v[...]
    ends = [offs[j] for j in range(_B)]
    starts = [jnp.int32(0)] + ends[:-1]

    zero16 = jnp.zeros((16,), jnp.float32)
    for j in range(_B):
        for c in range(_NLANE):
            acc[j, pl.ds(c * 16, 16)] = zero16

    def issue(k, buf, sem):
        pltpu.async_copy(
            feat_hbm.at[pl.ds(base + k * _CHUNK, _CHUNK)], buf, sem)

    def wait(k, buf, sem):
        pltpu.make_async_copy(
            feat_hbm.at[pl.ds(base + k * _CHUNK, _CHUNK)], buf, sem).wait()

    def process(buf, k):
        cb = base + k * _CHUNK
        sf = sum(((ends[j] <= cb).astype(jnp.int32)) for j in range(_B))
        sl = sum(((ends[j] <= cb + _CHUNK - 1).astype(jnp.int32))
                 for j in range(_B))

        @pl.when(sf == sl)
        def _fast():
            init = tuple(zero16 for _ in range(_NLANE))

            def grp_body(g, vecs):
                rb = g * _G
                out = []
                for c in range(_NLANE):
                    s = vecs[c]
                    for t in range(_G):
                        s = s + buf[rb + t, pl.ds(c * 16, 16)]
                    out.append(s)
                return tuple(out)

            vecs = plsc.parallel_loop(
                0, _CHUNK // _G, carry=init)(grp_body)
            for j in range(_B):
                @pl.when(sf == j)
                def _flush(j=j, vecs=vecs):
                    for c in range(_NLANE):
                        plsc.addupdate(acc.at[j, pl.ds(c * 16, 16)], vecs[c])

        @pl.when(sf != sl)
        def _slow():
            for j in range(_B):
                lo = jnp.clip(starts[j] - cb, 0, _CHUNK)
                hi = jnp.clip(ends[j] - cb, 0, _CHUNK)

                @pl.when(hi > lo)
                def _seg(j=j, lo=lo, hi=hi):
                    def rem_body(r, c2):
                        for c in range(_NLANE):
                            plsc.addupdate(acc.at[j, pl.ds(c * 16, 16)],
                                           buf[r, pl.ds(c * 16, 16)])
                        return c2

                    lax.fori_loop(lo, hi, rem_body, 0)

    issue(0, buf_a, sem_a)
    issue(1, buf_b, sem_b)

    def pair_body(p, carry):
        k0 = 2 * p
        wait(k0, buf_a, sem_a)
        process(buf_a, k0)

        @pl.when(k0 + 2 < _NCHUNK)
        def _ra():
            issue(k0 + 2, buf_a, sem_a)

        wait(k0 + 1, buf_b, sem_b)
        process(buf_b, k0 + 1)

        @pl.when(k0 + 3 < _NCHUNK)
        def _rb():
            issue(k0 + 3, buf_b, sem_b)

        return carry

    lax.fori_loop(0, _NCHUNK // 2, pair_body, 0)

    pltpu.sync_copy(acc, out_hbm.at[wid])


def _sc_segsum(feat, offset):
    mesh = plsc.VectorSubcoreMesh(core_axis_name="c", subcore_axis_name="s")
    return pl.kernel(
        _sc_segsum_body,
        mesh=mesh,
        out_type=jax.ShapeDtypeStruct((_NW, _B, _C), jnp.float32),
        scratch_types=[
            pltpu.VMEM((_CHUNK, _C), jnp.float32),
            pltpu.VMEM((_CHUNK, _C), jnp.float32),
            pltpu.VMEM((_B, _C), jnp.float32),
            pltpu.VMEM((_B,), jnp.int32),
            pltpu.SemaphoreType.DMA,
            pltpu.SemaphoreType.DMA,
        ],
    )(feat, offset)


def _tc_partial_body(feat_ref, starts_ref, ends_ref, out_ref, *, nsteps):
    i = pl.program_id(0)

    @pl.when(i == 0)
    def _init():
        out_ref[...] = jnp.zeros_like(out_ref)

    rows = (lax.broadcasted_iota(jnp.int32, (_B, _TBLK), 1)
            + (i + _S_SC // _TBLK) * _TBLK)
    onehot_t = ((rows >= starts_ref[...]) & (rows < ends_ref[...])
                ).astype(jnp.float32)
    out_ref[...] += jnp.dot(onehot_t, feat_ref[...],
                            preferred_element_type=jnp.float32)


def _tc_partial(feat, starts2d, ends2d):
    nsteps = (_N - _S_SC) // _TBLK
    import functools
    body = functools.partial(_tc_partial_body, nsteps=nsteps)
    return pl.pallas_call(
        body,
        grid=(nsteps,),
        in_specs=[
            pl.BlockSpec((_TBLK, _C), lambda i: (i + _S_SC // _TBLK, 0)),
            pl.BlockSpec((_B, 1), lambda i: (0, 0)),
            pl.BlockSpec((_B, 1), lambda i: (0, 0)),
        ],
        out_specs=pl.BlockSpec((_B, _C), lambda i: (0, 0)),
        out_shape=jax.ShapeDtypeStruct((_B, _C), jnp.float32),
        compiler_params=pltpu.CompilerParams(
            dimension_semantics=("arbitrary",)),
    )(feat, starts2d, ends2d)


def _head_body(part_ref, tcp_ref, invc_ref, w1_ref, bn1s_ref, bn1b_ref,
               w2_ref, b2_ref, w3_ref, b3_ref, out_ref):
    rows = lax.broadcasted_iota(jnp.int32, (_B, _NW * _B), 1)
    cols = lax.broadcasted_iota(jnp.int32, (_B, _NW * _B), 0)
    sel = (rows % _B == cols).astype(jnp.float32)
    seg_sum = (jnp.dot(sel, part_ref[...], preferred_element_type=jnp.float32)
               + tcp_ref[...])
    gf = seg_sum * invc_ref[...]
    h = jnp.dot(gf, w1_ref[...], preferred_element_type=jnp.float32)
    h = h * bn1s_ref[...] + bn1b_ref[...]
    h = jnp.maximum(h, 0.0)
    h = jnp.dot(h, w2_ref[...], preferred_element_type=jnp.float32) + b2_ref[...]
    h = jnp.maximum(h, 0.0)
    out_ref[...] = (jnp.dot(h, w3_ref[...], preferred_element_type=jnp.float32)
                    + b3_ref[...])


def kernel(feat, offset, W1, b1, gamma, beta, rmean, rvar, W2, b2, W3, b3):
    offset = offset.astype(jnp.int32)
    starts = jnp.concatenate([jnp.zeros((1,), jnp.int32), offset[:-1]])
    counts = jnp.maximum((offset - starts).astype(jnp.float32), 1.0)
    invc = (1.0 / counts).reshape(_B, 1)
    bn1s = gamma * lax.rsqrt(rvar + 1e-5)
    bn1b = (b1 - rmean) * bn1s + beta

    partials = _sc_segsum(feat, offset).reshape(_NW * _B, _C)
    tcp = _tc_partial(feat, starts.reshape(_B, 1), offset.reshape(_B, 1))

    out = pl.pallas_call(
        _head_body,
        in_specs=[
            pl.BlockSpec((_NW * _B, _C), lambda: (0, 0)),
            pl.BlockSpec((_B, _C), lambda: (0, 0)),
            pl.BlockSpec((_B, 1), lambda: (0, 0)),
            pl.BlockSpec((_C, _H), lambda: (0, 0)),
            pl.BlockSpec((1, _H), lambda: (0, 0)),
            pl.BlockSpec((1, _H), lambda: (0, 0)),
            pl.BlockSpec((_H, _H), lambda: (0, 0)),
            pl.BlockSpec((1, _H), lambda: (0, 0)),
            pl.BlockSpec((_H, _KOUT), lambda: (0, 0)),
            pl.BlockSpec((1, _KOUT), lambda: (0, 0)),
        ],
        out_specs=pl.BlockSpec((_B, _KOUT), lambda: (0, 0)),
        out_shape=jax.ShapeDtypeStruct((_B, _KOUT), jnp.float32),
    )(partials, tcp, invc,
      W1, bn1s.reshape(1, _H), bn1b.reshape(1, _H),
      W2, b2.reshape(1, _H), W3, b3.reshape(1, _KOUT))
    return out.reshape(_B, 6, 3)
